# baseline (speedup 1.0000x reference)
import functools

import jax
import jax.numpy as jnp
from jax import lax
from jax.experimental import pallas as pl
from jax.experimental.pallas import tpu as pltpu
from jax.experimental.pallas import tpu_sc as plsc

NNODES = 10000
NEDGES = 320000
DF = 128
F2 = DF // 2
CH = 40
NC = 2
NS = 16
NW = NC * NS
K = 125
DCH = NEDGES // (NW * K)
SCH = NEDGES // (NS * K)
NB = 5
AHEAD = 2
RBIG = 632
RLAST = NNODES - (NS - 1) * RBIG
WDEG = 16

_mesh = plsc.VectorSubcoreMesh(
    core_axis_name="c", subcore_axis_name="s", num_cores=NC, num_subcores=NS)
_sc_params = pltpu.CompilerParams(use_tc_tiling_on_sc=False)


def _split_copy(s, src_at, dst_at):
    base = s * RBIG

    @pl.when(s < NS - 1)
    def _():
        pltpu.sync_copy(src_at(base, RBIG), dst_at(base, RBIG))

    @pl.when(s == NS - 1)
    def _():
        pltpu.sync_copy(src_at(base, RLAST), dst_at(base, RLAST))


@functools.partial(
    pl.kernel,
    out_type=jax.ShapeDtypeStruct((NC, NNODES, WDEG), jnp.float32),
    mesh=_mesh,
    compiler_params=_sc_params,
    scratch_types=[
        pltpu.VMEM((DCH, K), jnp.int32),
        pltpu.VMEM((K, WDEG), jnp.float32),
        pltpu.VMEM_SHARED((NNODES, WDEG), jnp.float32),
    ],
)
def _deg_sc(edges_hbm, zeros16_hbm, ones16_hbm, out_hbm, dst_v, ones_v, deg_sp):
    c = lax.axis_index("c")
    s = lax.axis_index("s")
    _split_copy(s, lambda b, n: zeros16_hbm.at[pl.ds(b, n)],
                lambda b, n: deg_sp.at[pl.ds(b, n)])
    pltpu.sync_copy(ones16_hbm, ones_v)
    pltpu.sync_copy(edges_hbm.at[1, s, pl.ds(c * DCH, DCH)], dst_v)
    plsc.subcore_barrier()

    def body(j, _):
        pltpu.sync_copy(ones_v, deg_sp.at[dst_v.at[j]], add=True)
        return 0

    lax.fori_loop(0, DCH, body, 0)
    plsc.subcore_barrier()
    _split_copy(s, lambda b, n: deg_sp.at[pl.ds(b, n)],
                lambda b, n: out_hbm.at[c, pl.ds(b, n)])


FQ = DF // 4


@functools.partial(
    pl.kernel,
    out_type=jax.ShapeDtypeStruct((NNODES, 4, FQ), jnp.float32),
    mesh=_mesh,
    compiler_params=_sc_params,
    scratch_types=[
        pltpu.VMEM((SCH, K), jnp.int32),
        pltpu.VMEM((SCH, K), jnp.int32),
        pltpu.VMEM((NB, K, 1, FQ), jnp.float32),
        pltpu.VMEM_SHARED((NNODES, 1, FQ), jnp.float32),
        pltpu.VMEM_SHARED((NNODES, 1, FQ), jnp.float32),
        pltpu.SemaphoreType.DMA((NB,)),
        pltpu.SemaphoreType.DMA((NB,)),
    ],
)
def _spmm_sc(y_hbm, edges_hbm, out_hbm, src_v, dst_v, bufs, y_sp, z_sp,
             gsem, ssem):
    c = lax.axis_index("c")
    s = lax.axis_index("s")

    pltpu.sync_copy(edges_hbm.at[0, s], src_v)
    pltpu.sync_copy(edges_hbm.at[1, s], dst_v)

    for p in range(2):
        q = 2 * c + p
        _split_copy(s, lambda b, n: y_hbm.at[pl.ds(b, n), pl.ds(q, 1)],
                    lambda b, n: y_sp.at[pl.ds(b, n)])
        _split_copy(s, lambda b, n: y_sp.at[pl.ds(b, n)],
                    lambda b, n: z_sp.at[pl.ds(b, n)])
        plsc.subcore_barrier()

        for b in range(AHEAD):
            pltpu.async_copy(y_sp.at[src_v.at[b]], bufs.at[b], gsem.at[b])

        def body(g, _):
            for b in range(NB):
                j = NB * g + b
                pltpu.make_async_copy(
                    y_sp.at[src_v.at[0]], bufs.at[b], gsem.at[b]).wait()
                pltpu.async_copy(
                    bufs.at[b], z_sp.at[dst_v.at[j]], ssem.at[b], add=True)
                bn = (b + AHEAD) % NB
                jw = j + AHEAD - NB
                jn = j + AHEAD

                @pl.when(jn < SCH)
                def _(b=b, bn=bn, jw=jw, jn=jn):
                    @pl.when(jw >= 0)
                    def _():
                        pltpu.make_async_copy(
                            bufs.at[bn], z_sp.at[dst_v.at[0]],
                            ssem.at[bn]).wait()

                    pltpu.async_copy(
                        y_sp.at[src_v.at[jn]], bufs.at[bn], gsem.at[bn])
            return 0

        lax.fori_loop(0, SCH // NB, body, 0)
        for b in range(NB):
            pltpu.make_async_copy(
                bufs.at[b], z_sp.at[dst_v.at[0]], ssem.at[b]).wait()
        plsc.subcore_barrier()
        _split_copy(s, lambda b, n: z_sp.at[pl.ds(b, n)],
                    lambda b, n: out_hbm.at[pl.ds(b, n), pl.ds(q, 1)])


_GRID = 5
_RB = NNODES // _GRID


def _dinv_of(deg_ref):
    dtot = deg_ref[0, :, 0:1] + deg_ref[1, :, 0:1] + 1.0
    return lax.rsqrt(dtot)


def _mm(a, b):
    return jax.lax.dot_general(a, b, (((1,), (0,)), ((), ())),
                               preferred_element_type=jnp.float32,
                               precision=lax.Precision.HIGHEST)


def _b1_body(x_ref, w_ref, deg_ref, y_ref):
    y_ref[...] = _mm(x_ref[...], w_ref[...]) * _dinv_of(deg_ref)


def _b2_body(z_ref, deg_ref, w_ref, b1_ref, y_ref):
    dinv = _dinv_of(deg_ref)
    h = jnp.maximum(z_ref[...] * dinv + b1_ref[...], 0.0)
    y_ref[...] = _mm(h, w_ref[...]) * dinv


def _b3_body(z_ref, deg_ref, w_ref, b2_ref, bc_ref, o_ref):
    dinv = _dinv_of(deg_ref)
    h = z_ref[...] * dinv + b2_ref[...]
    logits = _mm(h, w_ref[...]) + bc_ref[...]
    m = jnp.max(logits, axis=1, keepdims=True)
    sh = logits - m
    out = sh - jnp.log(jnp.sum(jnp.exp(sh), axis=1, keepdims=True))
    o_ref[...] = out[:, :CH]


_row_spec = pl.BlockSpec((_RB, DF), lambda i: (i, 0))
_deg_spec = pl.BlockSpec((NC, _RB, WDEG), lambda i: (0, i, 0))
_w_spec = pl.BlockSpec((DF, DF), lambda i: (0, 0))
_b_spec = pl.BlockSpec((1, DF), lambda i: (0, 0))
_out_full = jax.ShapeDtypeStruct((NNODES, DF), jnp.float32)

_b1_call = pl.pallas_call(
    _b1_body, grid=(_GRID,),
    in_specs=[_row_spec, _w_spec, _deg_spec],
    out_specs=_row_spec, out_shape=_out_full)

_b2_call = pl.pallas_call(
    _b2_body, grid=(_GRID,),
    in_specs=[_row_spec, _deg_spec, _w_spec, _b_spec],
    out_specs=_row_spec, out_shape=_out_full)

_b3_call = pl.pallas_call(
    _b3_body, grid=(_GRID,),
    in_specs=[_row_spec, _deg_spec, _w_spec, _b_spec, _b_spec],
    out_specs=pl.BlockSpec((_RB, CH), lambda i: (i, 0)),
    out_shape=jax.ShapeDtypeStruct((NNODES, CH), jnp.float32))


def kernel(x, edge_index, W1, b1, W2, b2, Wc, bc):
    e4 = edge_index.astype(jnp.int32).reshape(2, NS, SCH, K)

    zeros16 = jnp.zeros((NNODES, WDEG), jnp.float32)
    ones16 = jnp.ones((K, WDEG), jnp.float32)
    b1r = b1.reshape(1, DF)
    b2r = b2.reshape(1, DF)
    wcp = jnp.zeros((DF, DF), jnp.float32).at[:, :CH].set(Wc)
    bcp = jnp.full((1, DF), -1e30, jnp.float32).at[0, :CH].set(bc)

    deg2 = _deg_sc(e4, zeros16, ones16)
    y1 = _b1_call(x, W1, deg2)
    z1 = _spmm_sc(y1.reshape(NNODES, 4, FQ), e4).reshape(NNODES, DF)
    y2 = _b2_call(z1, deg2, W2, b1r)
    z2 = _spmm_sc(y2.reshape(NNODES, 4, FQ), e4).reshape(NNODES, DF)
    return _b3_call(z2, deg2, wcp, b2r, bcp)


# speedup vs baseline: 2.0894x; 2.0894x over previous
import functools

import jax
import jax.numpy as jnp
from jax import lax
from jax.experimental import pallas as pl
from jax.experimental.pallas import tpu as pltpu
from jax.experimental.pallas import tpu_sc as plsc

NNODES = 10000
NEDGES = 320000
DF = 128
F2 = DF // 2
CH = 40
NC = 2
NS = 16
NW = NC * NS
K = 125
DCH = NEDGES // (NW * K)
SCH = NEDGES // (NS * K)
NB = 5
AHEAD = 2
RBIG = 632
RLAST = NNODES - (NS - 1) * RBIG
WDEG = 16

_mesh = plsc.VectorSubcoreMesh(
    core_axis_name="c", subcore_axis_name="s", num_cores=NC, num_subcores=NS)
_sc_params = pltpu.CompilerParams(use_tc_tiling_on_sc=False)


def _split_copy(s, src_at, dst_at):
    base = s * RBIG

    @pl.when(s < NS - 1)
    def _():
        pltpu.sync_copy(src_at(base, RBIG), dst_at(base, RBIG))

    @pl.when(s == NS - 1)
    def _():
        pltpu.sync_copy(src_at(base, RLAST), dst_at(base, RLAST))


@functools.partial(
    pl.kernel,
    out_type=jax.ShapeDtypeStruct((NC, NNODES, WDEG), jnp.float32),
    mesh=_mesh,
    compiler_params=_sc_params,
    scratch_types=[
        pltpu.VMEM((DCH, K), jnp.int32),
        pltpu.VMEM((K, WDEG), jnp.float32),
        pltpu.VMEM_SHARED((NNODES, WDEG), jnp.float32),
    ],
)
def _deg_sc(edges_hbm, zeros16_hbm, ones16_hbm, out_hbm, dst_v, ones_v, deg_sp):
    c = lax.axis_index("c")
    s = lax.axis_index("s")
    _split_copy(s, lambda b, n: zeros16_hbm.at[pl.ds(b, n)],
                lambda b, n: deg_sp.at[pl.ds(b, n)])
    pltpu.sync_copy(ones16_hbm, ones_v)
    pltpu.sync_copy(edges_hbm.at[1, s, pl.ds(c * DCH, DCH)], dst_v)
    plsc.subcore_barrier()

    def body(j, _):
        pltpu.sync_copy(ones_v, deg_sp.at[dst_v.at[j]], add=True)
        return 0

    lax.fori_loop(0, DCH, body, 0)
    plsc.subcore_barrier()
    _split_copy(s, lambda b, n: deg_sp.at[pl.ds(b, n)],
                lambda b, n: out_hbm.at[c, pl.ds(b, n)])


@functools.partial(
    pl.kernel,
    out_type=[jax.ShapeDtypeStruct((NNODES, F2), jnp.float32),
              jax.ShapeDtypeStruct((NNODES, F2), jnp.float32)],
    mesh=_mesh,
    compiler_params=_sc_params,
    scratch_types=[
        pltpu.VMEM((SCH, K), jnp.int32),
        pltpu.VMEM((SCH, K), jnp.int32),
        pltpu.VMEM((NB, K, F2), jnp.float32),
        pltpu.VMEM_SHARED((NNODES, F2), jnp.float32),
        pltpu.SemaphoreType.DMA((NB,)),
        pltpu.SemaphoreType.DMA((NB,)),
    ],
)
def _spmm_sc(ya_hbm, yb_hbm, edges_hbm, outa_hbm, outb_hbm,
             src_v, dst_v, bufs, z_sp, gsem, ssem):
    c = lax.axis_index("c")
    s = lax.axis_index("s")

    pltpu.sync_copy(edges_hbm.at[0, s], src_v)
    pltpu.sync_copy(edges_hbm.at[1, s], dst_v)

    def run_half(y_hbm, out_hbm):
        _split_copy(s, lambda b, n: y_hbm.at[pl.ds(b, n)],
                    lambda b, n: z_sp.at[pl.ds(b, n)])
        plsc.subcore_barrier()

        for b in range(AHEAD):
            pltpu.async_copy(y_hbm.at[src_v.at[b]], bufs.at[b], gsem.at[b])

        def body(g, _):
            for b in range(NB):
                j = NB * g + b
                pltpu.make_async_copy(
                    y_hbm.at[src_v.at[0]], bufs.at[b], gsem.at[b]).wait()
                pltpu.async_copy(
                    bufs.at[b], z_sp.at[dst_v.at[j]], ssem.at[b], add=True)
                bn = (b + AHEAD) % NB
                jw = j + AHEAD - NB
                jn = j + AHEAD

                @pl.when(jn < SCH)
                def _(b=b, bn=bn, jw=jw, jn=jn):
                    @pl.when(jw >= 0)
                    def _():
                        pltpu.make_async_copy(
                            bufs.at[bn], z_sp.at[dst_v.at[0]],
                            ssem.at[bn]).wait()

                    pltpu.async_copy(
                        y_hbm.at[src_v.at[jn]], bufs.at[bn], gsem.at[bn])
            return 0

        lax.fori_loop(0, SCH // NB, body, 0)
        for b in range(NB):
            pltpu.make_async_copy(
                bufs.at[b], z_sp.at[dst_v.at[0]], ssem.at[b]).wait()
        plsc.subcore_barrier()
        _split_copy(s, lambda b, n: z_sp.at[pl.ds(b, n)],
                    lambda b, n: out_hbm.at[pl.ds(b, n)])

    @pl.when(c == 0)
    def _():
        run_half(ya_hbm, outa_hbm)

    @pl.when(c == 1)
    def _():
        run_half(yb_hbm, outb_hbm)


_GRID = 5
_RB = NNODES // _GRID


def _dinv_of(deg_ref):
    dtot = deg_ref[0, :, 0:1] + deg_ref[1, :, 0:1] + 1.0
    return lax.rsqrt(dtot)


def _mm(a, b):
    return jax.lax.dot_general(a, b, (((1,), (0,)), ((), ())),
                               preferred_element_type=jnp.float32,
                               precision=lax.Precision.HIGHEST)


def _b1_body(x_ref, w_ref, deg_ref, ya_ref, yb_ref):
    y = _mm(x_ref[...], w_ref[...]) * _dinv_of(deg_ref)
    ya_ref[...] = y[:, :F2]
    yb_ref[...] = y[:, F2:]


def _b2_body(za_ref, zb_ref, deg_ref, w_ref, b1_ref, ya_ref, yb_ref):
    dinv = _dinv_of(deg_ref)
    z = jnp.concatenate([za_ref[...], zb_ref[...]], axis=1)
    h = jnp.maximum(z * dinv + b1_ref[...], 0.0)
    y = _mm(h, w_ref[...]) * dinv
    ya_ref[...] = y[:, :F2]
    yb_ref[...] = y[:, F2:]


def _b3_body(za_ref, zb_ref, deg_ref, w_ref, b2_ref, bc_ref, o_ref):
    dinv = _dinv_of(deg_ref)
    z = jnp.concatenate([za_ref[...], zb_ref[...]], axis=1)
    h = z * dinv + b2_ref[...]
    logits = _mm(h, w_ref[...]) + bc_ref[...]
    m = jnp.max(logits, axis=1, keepdims=True)
    sh = logits - m
    out = sh - jnp.log(jnp.sum(jnp.exp(sh), axis=1, keepdims=True))
    o_ref[...] = out[:, :CH]


_row_spec = pl.BlockSpec((_RB, DF), lambda i: (i, 0))
_pk_spec = pl.BlockSpec((_RB, F2), lambda i: (i, 0))
_deg_spec = pl.BlockSpec((NC, _RB, WDEG), lambda i: (0, i, 0))
_w_spec = pl.BlockSpec((DF, DF), lambda i: (0, 0))
_b_spec = pl.BlockSpec((1, DF), lambda i: (0, 0))
_out_pk = [jax.ShapeDtypeStruct((NNODES, F2), jnp.float32)] * 2

_b1_call = pl.pallas_call(
    _b1_body, grid=(_GRID,),
    in_specs=[_row_spec, _w_spec, _deg_spec],
    out_specs=[_pk_spec, _pk_spec], out_shape=_out_pk)

_b2_call = pl.pallas_call(
    _b2_body, grid=(_GRID,),
    in_specs=[_pk_spec, _pk_spec, _deg_spec, _w_spec, _b_spec],
    out_specs=[_pk_spec, _pk_spec], out_shape=_out_pk)

_b3_call = pl.pallas_call(
    _b3_body, grid=(_GRID,),
    in_specs=[_pk_spec, _pk_spec, _deg_spec, _w_spec, _b_spec, _b_spec],
    out_specs=pl.BlockSpec((_RB, CH), lambda i: (i, 0)),
    out_shape=jax.ShapeDtypeStruct((NNODES, CH), jnp.float32))


def kernel(x, edge_index, W1, b1, W2, b2, Wc, bc):
    e4 = edge_index.astype(jnp.int32).reshape(2, NS, SCH, K)

    zeros16 = jnp.zeros((NNODES, WDEG), jnp.float32)
    ones16 = jnp.ones((K, WDEG), jnp.float32)
    b1r = b1.reshape(1, DF)
    b2r = b2.reshape(1, DF)
    wcp = jnp.zeros((DF, DF), jnp.float32).at[:, :CH].set(Wc)
    bcp = jnp.full((1, DF), -1e30, jnp.float32).at[0, :CH].set(bc)

    deg2 = _deg_sc(e4, zeros16, ones16)
    y1a, y1b = _b1_call(x, W1, deg2)
    z1a, z1b = _spmm_sc(y1a, y1b, e4)
    y2a, y2b = _b2_call(z1a, z1b, deg2, W2, b1r)
    z2a, z2b = _spmm_sc(y2a, y2b, e4)
    return _b3_call(z2a, z2b, deg2, wcp, b2r, bcp)
